# baseline (speedup 1.0000x reference)
import jax
import jax.numpy as jnp
from jax import lax
from jax.experimental import pallas as pl
from jax.experimental.pallas import tpu as pltpu
from jax.experimental.pallas import tpu_sc as plsc

N = 10000
E = 160000
D = 16
OUT = 16
H = 32

NC = 2
NS = 16
NW = NC * NS
CH = 125
NCHUNK = 40
EPT = CH * NCHUNK
EP8 = EPT // 8
EPK = E // 8
NROW = 640
NPAD = NROW * NS
NPK = NPAD // 8
DEPTH = 16


def _mesh():
    return plsc.VectorSubcoreMesh(
        core_axis_name="c", subcore_axis_name="s", num_cores=NC, num_subcores=NS)


def _gather_body(x_hbm, src_hbm, xs_hbm, idx_v, rows_v, sem, xsm):
    c = lax.axis_index("c")
    s = lax.axis_index("s")
    w = s * NC + c

    @pl.when(s == 0)
    def _():
        pltpu.sync_copy(x_hbm, xsm)
    pltpu.sync_copy(src_hbm.at[w], idx_v)
    plsc.subcore_barrier()

    def fire(j):
        pltpu.async_copy(xsm.at[idx_v.at[j]],
                         rows_v.at[pl.ds(j * CH, CH)], sem)

    def drain(j):
        pltpu.make_async_copy(xsm.at[idx_v.at[j]],
                              rows_v.at[pl.ds(j * CH, CH)], sem).wait()

    lax.fori_loop(0, DEPTH, lambda j, v: (fire(j), v)[1], 0)

    def steady(j, v):
        @pl.when(j + DEPTH < NCHUNK)
        def _():
            fire(j + DEPTH)
        drain(j)
        return v

    lax.fori_loop(0, NCHUNK, steady, 0)
    pltpu.sync_copy(rows_v, xs_hbm.at[w])


@jax.jit
def _sc_gather(x, src3):
    k = pl.kernel(
        _gather_body,
        out_type=jax.ShapeDtypeStruct((NW, EPT, D), jnp.float32),
        mesh=_mesh(),
        compiler_params=pltpu.CompilerParams(use_tc_tiling_on_sc=False),
        scratch_types=[
            pltpu.VMEM((NCHUNK, CH), jnp.int32),
            pltpu.VMEM((EPT, D), jnp.float32),
            pltpu.SemaphoreType.DMA,
            pltpu.VMEM_SHARED((N, D), jnp.float32),
        ],
    )
    return k(x, src3)


def _scatter_body(msg_hbm, dst_hbm, zeros_hbm, part_hbm, idx_v, msg_v, sem, acc):
    c = lax.axis_index("c")
    s = lax.axis_index("s")
    w = s * NC + c
    pltpu.sync_copy(zeros_hbm, acc.at[pl.ds(s * NROW, NROW)])
    pltpu.sync_copy(dst_hbm.at[w], idx_v)
    pltpu.sync_copy(msg_hbm.at[w], msg_v)
    plsc.subcore_barrier()

    def fire(j):
        pltpu.async_copy(msg_v.at[pl.ds(j * CH, CH)],
                         acc.at[idx_v.at[j]], sem, add=True)

    def drain(j):
        pltpu.make_async_copy(msg_v.at[pl.ds(j * CH, CH)],
                              acc.at[idx_v.at[j]], sem).wait()

    lax.fori_loop(0, DEPTH, lambda j, v: (fire(j), v)[1], 0)

    def steady(j, v):
        @pl.when(j + DEPTH < NCHUNK)
        def _():
            fire(j + DEPTH)
        drain(j)
        return v

    lax.fori_loop(0, NCHUNK, steady, 0)
    plsc.subcore_barrier()
    pltpu.sync_copy(acc.at[pl.ds(s * NROW, NROW)],
                    part_hbm.at[c, pl.ds(s * NROW, NROW)])


@jax.jit
def _sc_scatter(msg, dst3, zeros_slab):
    k = pl.kernel(
        _scatter_body,
        out_type=jax.ShapeDtypeStruct((NC, NPAD, D), jnp.float32),
        mesh=_mesh(),
        compiler_params=pltpu.CompilerParams(use_tc_tiling_on_sc=False),
        scratch_types=[
            pltpu.VMEM((NCHUNK, CH), jnp.int32),
            pltpu.VMEM((EPT, D), jnp.float32),
            pltpu.SemaphoreType.DMA,
            pltpu.VMEM_SHARED((NPAD, D), jnp.float32),
        ],
    )
    return k(msg, dst3, zeros_slab)


BE = 8000


def _edge_body(ea_ref, xs_ref, w1r_ref, s_ref, w2f_ref, msg_ref):
    t = jnp.maximum(jnp.dot(ea_ref[...], w1r_ref[...],
                            preferred_element_type=jnp.float32), 0.0)
    xss = jnp.dot(xs_ref[...], s_ref[...], preferred_element_type=jnp.float32)
    msg_ref[...] = jnp.dot(t * xss, w2f_ref[...],
                           preferred_element_type=jnp.float32)


def _tc_edge(ea, xs, w1r, smat, w2f):
    nblk = E // BE
    return pl.pallas_call(
        _edge_body,
        grid=(nblk,),
        in_specs=[
            pl.BlockSpec((BE, D), lambda i: (i, 0)),
            pl.BlockSpec((BE, D), lambda i: (i, 0)),
            pl.BlockSpec((D, D * OUT), lambda i: (0, 0)),
            pl.BlockSpec((D, D * OUT), lambda i: (0, 0)),
            pl.BlockSpec((D * OUT, OUT), lambda i: (0, 0)),
        ],
        out_specs=pl.BlockSpec((BE, D), lambda i: (i, 0)),
        out_shape=jax.ShapeDtypeStruct((E, OUT), jnp.float32),
    )(ea, xs, w1r, smat, w2f)


H8 = 8 * H


def _node_body(xp_ref, part_ref, eps_ref, w1_ref, b1_ref, w2_ref, b2_ref,
               mu_ref, g_ref, b_ref, out_ref):
    hf = (1.0 + eps_ref[0]) * (xp_ref[...] + part_ref[0] + part_ref[1])
    z = jnp.dot(hf, w1_ref[...], preferred_element_type=jnp.float32) + b1_ref[...]
    z = jnp.where(z > 0, z, 0.01 * z)
    z = jnp.dot(z, w2_ref[...], preferred_element_type=jnp.float32) + b2_ref[...]
    mu = jnp.dot(z, mu_ref[...], preferred_element_type=jnp.float32)
    zc = z - mu
    var = jnp.dot(zc * zc, mu_ref[...], preferred_element_type=jnp.float32)
    out_ref[...] = zc * jax.lax.rsqrt(var + 1e-5) * g_ref[...] + b_ref[...]


def _tc_node(xp, part_p, eps, w1blk, b1t, w2blk, b2t, mublk, g8, b8):
    return pl.pallas_call(
        _node_body,
        grid=(1,),
        in_specs=[
            pl.BlockSpec((NPK, 128), lambda i: (0, 0)),
            pl.BlockSpec((NC, NPK, 128), lambda i: (0, 0, 0)),
            pl.BlockSpec((1,), lambda i: (0,)),
            pl.BlockSpec((128, H8), lambda i: (0, 0)),
            pl.BlockSpec((1, H8), lambda i: (0, 0)),
            pl.BlockSpec((H8, H8), lambda i: (0, 0)),
            pl.BlockSpec((1, H8), lambda i: (0, 0)),
            pl.BlockSpec((H8, H8), lambda i: (0, 0)),
            pl.BlockSpec((1, H8), lambda i: (0, 0)),
            pl.BlockSpec((1, H8), lambda i: (0, 0)),
        ],
        out_specs=pl.BlockSpec((NPK, H8), lambda i: (0, 0)),
        out_shape=jax.ShapeDtypeStruct((NPK, H8), jnp.float32),
    )(xp, part_p, eps, w1blk, b1t, w2blk, b2t, mublk, g8, b8)


def kernel(x, edge_index, edge_attr, eps, W_e1, W_e2, mlp_W1, mlp_b1,
           mlp_W2, mlp_b2, ln_g, ln_b):
    src3 = edge_index[0].reshape(NW, NCHUNK, CH)
    dst3 = edge_index[1].reshape(NW, NCHUNK, CH)

    eye = jnp.eye(D, dtype=jnp.float32)
    eye8 = jnp.eye(8, dtype=jnp.float32)
    rmat = jnp.repeat(eye, D, axis=1)
    smat = jnp.tile(eye, (1, D))
    w1r = W_e1 @ rmat
    w2f = W_e2.reshape(D * D, OUT)
    zeros_slab = jnp.zeros((NROW, D), jnp.float32)
    nw1blk = jnp.kron(eye8, mlp_W1)
    nw2blk = jnp.kron(eye8, mlp_W2)
    mublk = jnp.kron(eye8, jnp.full((H, H), 1.0 / H, jnp.float32))
    b1t = jnp.tile(mlp_b1, 8).reshape(1, H8)
    b2t = jnp.tile(mlp_b2, 8).reshape(1, H8)
    g8 = jnp.tile(ln_g, 8).reshape(1, H8)
    b8 = jnp.tile(ln_b, 8).reshape(1, H8)
    xp = jnp.concatenate(
        [x.reshape(N * D // 128, 128),
         jnp.zeros((NPK - N * D // 128, 128), jnp.float32)])

    xs = _sc_gather(x, src3).reshape(E, D)
    msg = _tc_edge(edge_attr, xs, w1r, smat, w2f)
    part_p = _sc_scatter(msg.reshape(NW, EPT, D), dst3,
                         zeros_slab).reshape(NC, NPK, 128)

    out_p = _tc_node(xp, part_p, eps.astype(jnp.float32),
                     nw1blk, b1t, nw2blk, b2t, mublk, g8, b8)
    return out_p.reshape(NPAD, H)[:N]


# speedup vs baseline: 1.3758x; 1.3758x over previous
import jax
import jax.numpy as jnp
from jax import lax
from jax.experimental import pallas as pl
from jax.experimental.pallas import tpu as pltpu
from jax.experimental.pallas import tpu_sc as plsc

N = 10000
E = 160000
D = 16
OUT = 16
H = 32

NC = 2
NS = 16
NW = NC * NS
CH = 125
NCHUNK = 40
EPT = CH * NCHUNK
EP8 = EPT // 8
EPK = E // 8
NROW = 640
NPAD = NROW * NS
NPK = NPAD // 8
DEPTH = 16


def _mesh():
    return plsc.VectorSubcoreMesh(
        core_axis_name="c", subcore_axis_name="s", num_cores=NC, num_subcores=NS)


def _gather_body(x_hbm, src_hbm, xs_hbm, idx_v, rows_v, sem, xsm):
    c = lax.axis_index("c")
    s = lax.axis_index("s")
    w = s * NC + c

    @pl.when(s == 0)
    def _():
        pltpu.sync_copy(x_hbm, xsm)
    pltpu.sync_copy(src_hbm.at[w], idx_v)
    plsc.subcore_barrier()

    def fire(j):
        pltpu.async_copy(xsm.at[idx_v.at[j]],
                         rows_v.at[pl.ds(j * CH, CH)], sem)

    def drain(j):
        pltpu.make_async_copy(xsm.at[idx_v.at[j]],
                              rows_v.at[pl.ds(j * CH, CH)], sem).wait()

    lax.fori_loop(0, DEPTH, lambda j, v: (fire(j), v)[1], 0)

    def steady(j, v):
        @pl.when(j + DEPTH < NCHUNK)
        def _():
            fire(j + DEPTH)
        drain(j)
        return v

    lax.fori_loop(0, NCHUNK, steady, 0)
    pltpu.sync_copy(rows_v, xs_hbm.at[w])


@jax.jit
def _sc_gather(x, src3):
    k = pl.kernel(
        _gather_body,
        out_type=jax.ShapeDtypeStruct((NW, EPT, D), jnp.float32),
        mesh=_mesh(),
        compiler_params=pltpu.CompilerParams(use_tc_tiling_on_sc=False),
        scratch_types=[
            pltpu.VMEM((NCHUNK, CH), jnp.int32),
            pltpu.VMEM((EPT, D), jnp.float32),
            pltpu.SemaphoreType.DMA,
            pltpu.VMEM_SHARED((N, D), jnp.float32),
        ],
    )
    return k(x, src3)


def _scatter_body(msg_hbm, dst_hbm, zeros_hbm, part_hbm, idx_v, msg_v, sem, acc):
    c = lax.axis_index("c")
    s = lax.axis_index("s")
    w = s * NC + c
    pltpu.sync_copy(zeros_hbm, acc.at[pl.ds(s * NROW, NROW)])
    pltpu.sync_copy(dst_hbm.at[w], idx_v)
    pltpu.sync_copy(msg_hbm.at[w], msg_v)
    plsc.subcore_barrier()

    def fire(j):
        pltpu.async_copy(msg_v.at[pl.ds(j * CH, CH)],
                         acc.at[idx_v.at[j]], sem, add=True)

    def drain(j):
        pltpu.make_async_copy(msg_v.at[pl.ds(j * CH, CH)],
                              acc.at[idx_v.at[j]], sem).wait()

    lax.fori_loop(0, DEPTH, lambda j, v: (fire(j), v)[1], 0)

    def steady(j, v):
        @pl.when(j + DEPTH < NCHUNK)
        def _():
            fire(j + DEPTH)
        drain(j)
        return v

    lax.fori_loop(0, NCHUNK, steady, 0)
    plsc.subcore_barrier()
    pltpu.sync_copy(acc.at[pl.ds(s * NROW, NROW)],
                    part_hbm.at[c, pl.ds(s * NROW, NROW)])


@jax.jit
def _sc_scatter(msg, dst3, zeros_slab):
    k = pl.kernel(
        _scatter_body,
        out_type=jax.ShapeDtypeStruct((NC, NPAD, D), jnp.float32),
        mesh=_mesh(),
        compiler_params=pltpu.CompilerParams(use_tc_tiling_on_sc=False),
        scratch_types=[
            pltpu.VMEM((NCHUNK, CH), jnp.int32),
            pltpu.VMEM((EPT, D), jnp.float32),
            pltpu.SemaphoreType.DMA,
            pltpu.VMEM_SHARED((NPAD, D), jnp.float32),
        ],
    )
    return k(msg, dst3, zeros_slab)


BR = 1000


def _edge_body(ea_ref, xs_ref, w1r_ref, s_ref, w2f_ref, msg_ref):
    ea = ea_ref[...].astype(jnp.bfloat16)
    xs = xs_ref[...].astype(jnp.bfloat16)
    t = jnp.maximum(jnp.dot(ea, w1r_ref[...],
                            preferred_element_type=jnp.float32), 0.0)
    xss = jnp.dot(xs, s_ref[...], preferred_element_type=jnp.float32)
    u = (t * xss).astype(jnp.bfloat16)
    msg_ref[...] = jnp.dot(u, w2f_ref[...], preferred_element_type=jnp.float32)


def _tc_edge(ea_p, xs_p, w1rblk, sblk, w2blk):
    nblk = EPK // BR
    return pl.pallas_call(
        _edge_body,
        grid=(nblk,),
        in_specs=[
            pl.BlockSpec((BR, 128), lambda i: (i, 0)),
            pl.BlockSpec((BR, 128), lambda i: (i, 0)),
            pl.BlockSpec((128, 8 * D * OUT), lambda i: (0, 0)),
            pl.BlockSpec((128, 8 * D * OUT), lambda i: (0, 0)),
            pl.BlockSpec((8 * D * OUT, 128), lambda i: (0, 0)),
        ],
        out_specs=pl.BlockSpec((BR, 128), lambda i: (i, 0)),
        out_shape=jax.ShapeDtypeStruct((EPK, 128), jnp.float32),
    )(ea_p, xs_p, w1rblk, sblk, w2blk)


H8 = 8 * H


def _node_body(xp_ref, part_ref, eps_ref, w1_ref, b1_ref, w2_ref, b2_ref,
               mu_ref, g_ref, b_ref, out_ref):
    hf = (1.0 + eps_ref[0]) * (xp_ref[...] + part_ref[0] + part_ref[1])
    z = jnp.dot(hf, w1_ref[...], preferred_element_type=jnp.float32) + b1_ref[...]
    z = jnp.where(z > 0, z, 0.01 * z)
    z = jnp.dot(z, w2_ref[...], preferred_element_type=jnp.float32) + b2_ref[...]
    mu = jnp.dot(z, mu_ref[...], preferred_element_type=jnp.float32)
    zc = z - mu
    var = jnp.dot(zc * zc, mu_ref[...], preferred_element_type=jnp.float32)
    out_ref[...] = zc * jax.lax.rsqrt(var + 1e-5) * g_ref[...] + b_ref[...]


def _tc_node(xp, part_p, eps, w1blk, b1t, w2blk, b2t, mublk, g8, b8):
    return pl.pallas_call(
        _node_body,
        grid=(1,),
        in_specs=[
            pl.BlockSpec((NPK, 128), lambda i: (0, 0)),
            pl.BlockSpec((NC, NPK, 128), lambda i: (0, 0, 0)),
            pl.BlockSpec((1,), lambda i: (0,)),
            pl.BlockSpec((128, H8), lambda i: (0, 0)),
            pl.BlockSpec((1, H8), lambda i: (0, 0)),
            pl.BlockSpec((H8, H8), lambda i: (0, 0)),
            pl.BlockSpec((1, H8), lambda i: (0, 0)),
            pl.BlockSpec((H8, H8), lambda i: (0, 0)),
            pl.BlockSpec((1, H8), lambda i: (0, 0)),
            pl.BlockSpec((1, H8), lambda i: (0, 0)),
        ],
        out_specs=pl.BlockSpec((NPK, H8), lambda i: (0, 0)),
        out_shape=jax.ShapeDtypeStruct((NPK, H8), jnp.float32),
    )(xp, part_p, eps, w1blk, b1t, w2blk, b2t, mublk, g8, b8)


def kernel(x, edge_index, edge_attr, eps, W_e1, W_e2, mlp_W1, mlp_b1,
           mlp_W2, mlp_b2, ln_g, ln_b):
    ea_p = edge_attr.reshape(EPK, 128)
    src3 = edge_index[0].reshape(NW, NCHUNK, CH)
    dst3 = edge_index[1].reshape(NW, NCHUNK, CH)

    eye = jnp.eye(D, dtype=jnp.float32)
    eye8 = jnp.eye(8, dtype=jnp.float32)
    rmat = jnp.repeat(eye, D, axis=1)
    smat = jnp.tile(eye, (1, D))
    w1rblk = jnp.kron(eye8, W_e1 @ rmat).astype(jnp.bfloat16)
    sblk = jnp.kron(eye8, smat).astype(jnp.bfloat16)
    w2blk = jnp.kron(eye8, W_e2.reshape(D * D, OUT)).astype(jnp.bfloat16)
    zeros_slab = jnp.zeros((NROW, D), jnp.float32)
    nw1blk = jnp.kron(eye8, mlp_W1)
    nw2blk = jnp.kron(eye8, mlp_W2)
    mublk = jnp.kron(eye8, jnp.full((H, H), 1.0 / H, jnp.float32))
    b1t = jnp.tile(mlp_b1, 8).reshape(1, H8)
    b2t = jnp.tile(mlp_b2, 8).reshape(1, H8)
    g8 = jnp.tile(ln_g, 8).reshape(1, H8)
    b8 = jnp.tile(ln_b, 8).reshape(1, H8)
    xp = jnp.concatenate(
        [x.reshape(N * D // 128, 128),
         jnp.zeros((NPK - N * D // 128, 128), jnp.float32)])

    xs_p = _sc_gather(x, src3).reshape(EPK, 128)
    msg_p = _tc_edge(ea_p, xs_p, w1rblk, sblk, w2blk)
    part_p = _sc_scatter(msg_p.reshape(NW, EPT, D), dst3,
                         zeros_slab).reshape(NC, NPK, 128)

    out_p = _tc_node(xp, part_p, eps.astype(jnp.float32),
                     nw1blk, b1t, nw2blk, b2t, mublk, g8, b8)
    return out_p.reshape(NPAD, H)[:N]


# speedup vs baseline: 1.5289x; 1.1113x over previous
import jax
import jax.numpy as jnp
from jax import lax
from jax.experimental import pallas as pl
from jax.experimental.pallas import tpu as pltpu
from jax.experimental.pallas import tpu_sc as plsc

N = 10000
E = 160000
D = 16
OUT = 16
H = 32

NC = 2
NS = 16
NW = NC * NS
CH = 125
NCHUNK = 40
EPT = CH * NCHUNK
EP8 = EPT // 8
EPK = E // 8
NROW = 640
NPAD = NROW * NS
NPK = NPAD // 8
DEPTH = 16


def _mesh():
    return plsc.VectorSubcoreMesh(
        core_axis_name="c", subcore_axis_name="s", num_cores=NC, num_subcores=NS)


def _gather_body(x_hbm, src_hbm, xs_hbm, idx_v, rows_v, sem, xsm):
    c = lax.axis_index("c")
    s = lax.axis_index("s")
    w = s * NC + c

    @pl.when(s == 0)
    def _():
        pltpu.sync_copy(x_hbm, xsm)
    pltpu.sync_copy(src_hbm.at[w], idx_v)
    plsc.subcore_barrier()

    def fire(j):
        pltpu.async_copy(xsm.at[idx_v.at[j]],
                         rows_v.at[pl.ds(j * CH, CH)], sem)

    def drain(j):
        pltpu.make_async_copy(xsm.at[idx_v.at[j]],
                              rows_v.at[pl.ds(j * CH, CH)], sem).wait()

    lax.fori_loop(0, DEPTH, lambda j, v: (fire(j), v)[1], 0)

    def steady(j, v):
        @pl.when(j + DEPTH < NCHUNK)
        def _():
            fire(j + DEPTH)
        drain(j)
        return v

    lax.fori_loop(0, NCHUNK, steady, 0)
    pltpu.sync_copy(rows_v, xs_hbm.at[w])


@jax.jit
def _sc_gather(x, src3):
    k = pl.kernel(
        _gather_body,
        out_type=jax.ShapeDtypeStruct((NW, EPT, D), jnp.float32),
        mesh=_mesh(),
        compiler_params=pltpu.CompilerParams(use_tc_tiling_on_sc=False),
        scratch_types=[
            pltpu.VMEM((NCHUNK, CH), jnp.int32),
            pltpu.VMEM((EPT, D), jnp.float32),
            pltpu.SemaphoreType.DMA,
            pltpu.VMEM_SHARED((N, D), jnp.float32),
        ],
    )
    return k(x, src3)


def _scatter_body(msg_hbm, dst_hbm, zeros_hbm, part_hbm, idx_v, msg_v, sem, acc):
    c = lax.axis_index("c")
    s = lax.axis_index("s")
    w = s * NC + c
    pltpu.sync_copy(zeros_hbm, acc.at[pl.ds(s * NROW, NROW)])
    pltpu.sync_copy(dst_hbm.at[w], idx_v)
    pltpu.sync_copy(msg_hbm.at[w], msg_v)
    plsc.subcore_barrier()

    def fire(j):
        pltpu.async_copy(msg_v.at[pl.ds(j * CH, CH)],
                         acc.at[idx_v.at[j]], sem, add=True)

    def drain(j):
        pltpu.make_async_copy(msg_v.at[pl.ds(j * CH, CH)],
                              acc.at[idx_v.at[j]], sem).wait()

    lax.fori_loop(0, DEPTH, lambda j, v: (fire(j), v)[1], 0)

    def steady(j, v):
        @pl.when(j + DEPTH < NCHUNK)
        def _():
            fire(j + DEPTH)
        drain(j)
        return v

    lax.fori_loop(0, NCHUNK, steady, 0)
    plsc.subcore_barrier()
    pltpu.sync_copy(acc.at[pl.ds(s * NROW, NROW)],
                    part_hbm.at[c, pl.ds(s * NROW, NROW)])


@jax.jit
def _sc_scatter(msg, dst3, zeros_slab):
    k = pl.kernel(
        _scatter_body,
        out_type=jax.ShapeDtypeStruct((NC, NPAD, D), jnp.float32),
        mesh=_mesh(),
        compiler_params=pltpu.CompilerParams(use_tc_tiling_on_sc=False),
        scratch_types=[
            pltpu.VMEM((NCHUNK, CH), jnp.int32),
            pltpu.VMEM((EPT, D), jnp.float32),
            pltpu.SemaphoreType.DMA,
            pltpu.VMEM_SHARED((NPAD, D), jnp.float32),
        ],
    )
    return k(msg, dst3, zeros_slab)


BR = 2000


def _edge_body(ea_ref, xs_ref, w1r_ref, s_ref, w2f_ref, msg_ref):
    ea = ea_ref[...].astype(jnp.bfloat16)
    xs = xs_ref[...].astype(jnp.bfloat16)
    for e8 in range(8):
        sl = slice(e8 * D, (e8 + 1) * D)
        t8 = jnp.maximum(jnp.dot(ea[:, sl], w1r_ref[...],
                                 preferred_element_type=jnp.float32), 0.0)
        x8 = jnp.dot(xs[:, sl], s_ref[...],
                     preferred_element_type=jnp.float32)
        u8 = (t8 * x8).astype(jnp.bfloat16)
        msg_ref[:, sl] = jnp.dot(u8, w2f_ref[...],
                                 preferred_element_type=jnp.float32)


def _tc_edge(ea_p, xs_p, w1r, smat, w2f):
    nblk = EPK // BR
    return pl.pallas_call(
        _edge_body,
        grid=(nblk,),
        in_specs=[
            pl.BlockSpec((BR, 128), lambda i: (i, 0)),
            pl.BlockSpec((BR, 128), lambda i: (i, 0)),
            pl.BlockSpec((D, D * OUT), lambda i: (0, 0)),
            pl.BlockSpec((D, D * OUT), lambda i: (0, 0)),
            pl.BlockSpec((D * OUT, OUT), lambda i: (0, 0)),
        ],
        out_specs=pl.BlockSpec((BR, 128), lambda i: (i, 0)),
        out_shape=jax.ShapeDtypeStruct((EPK, 128), jnp.float32),
    )(ea_p, xs_p, w1r, smat, w2f)


H8 = 8 * H


def _node_body(xp_ref, part_ref, eps_ref, w1_ref, b1_ref, w2_ref, b2_ref,
               mu_ref, g_ref, b_ref, out_ref):
    hf = (1.0 + eps_ref[0]) * (xp_ref[...] + part_ref[0] + part_ref[1])
    z = jnp.dot(hf, w1_ref[...], preferred_element_type=jnp.float32) + b1_ref[...]
    z = jnp.where(z > 0, z, 0.01 * z)
    z = jnp.dot(z, w2_ref[...], preferred_element_type=jnp.float32) + b2_ref[...]
    mu = jnp.dot(z, mu_ref[...], preferred_element_type=jnp.float32)
    zc = z - mu
    var = jnp.dot(zc * zc, mu_ref[...], preferred_element_type=jnp.float32)
    out_ref[...] = zc * jax.lax.rsqrt(var + 1e-5) * g_ref[...] + b_ref[...]


def _tc_node(xp, part_p, eps, w1blk, b1t, w2blk, b2t, mublk, g8, b8):
    return pl.pallas_call(
        _node_body,
        grid=(1,),
        in_specs=[
            pl.BlockSpec((NPK, 128), lambda i: (0, 0)),
            pl.BlockSpec((NC, NPK, 128), lambda i: (0, 0, 0)),
            pl.BlockSpec((1,), lambda i: (0,)),
            pl.BlockSpec((128, H8), lambda i: (0, 0)),
            pl.BlockSpec((1, H8), lambda i: (0, 0)),
            pl.BlockSpec((H8, H8), lambda i: (0, 0)),
            pl.BlockSpec((1, H8), lambda i: (0, 0)),
            pl.BlockSpec((H8, H8), lambda i: (0, 0)),
            pl.BlockSpec((1, H8), lambda i: (0, 0)),
            pl.BlockSpec((1, H8), lambda i: (0, 0)),
        ],
        out_specs=pl.BlockSpec((NPK, H8), lambda i: (0, 0)),
        out_shape=jax.ShapeDtypeStruct((NPK, H8), jnp.float32),
    )(xp, part_p, eps, w1blk, b1t, w2blk, b2t, mublk, g8, b8)


def kernel(x, edge_index, edge_attr, eps, W_e1, W_e2, mlp_W1, mlp_b1,
           mlp_W2, mlp_b2, ln_g, ln_b):
    ea_p = edge_attr.reshape(EPK, 128)
    src3 = edge_index[0].reshape(NW, NCHUNK, CH)
    dst3 = edge_index[1].reshape(NW, NCHUNK, CH)

    eye = jnp.eye(D, dtype=jnp.float32)
    eye8 = jnp.eye(8, dtype=jnp.float32)
    rmat = jnp.repeat(eye, D, axis=1)
    smat = jnp.tile(eye, (1, D))
    w1rb = (W_e1 @ rmat).astype(jnp.bfloat16)
    smatb = smat.astype(jnp.bfloat16)
    w2fb = W_e2.reshape(D * D, OUT).astype(jnp.bfloat16)
    zeros_slab = jnp.zeros((NROW, D), jnp.float32)
    nw1blk = jnp.kron(eye8, mlp_W1)
    nw2blk = jnp.kron(eye8, mlp_W2)
    mublk = jnp.kron(eye8, jnp.full((H, H), 1.0 / H, jnp.float32))
    b1t = jnp.tile(mlp_b1, 8).reshape(1, H8)
    b2t = jnp.tile(mlp_b2, 8).reshape(1, H8)
    g8 = jnp.tile(ln_g, 8).reshape(1, H8)
    b8 = jnp.tile(ln_b, 8).reshape(1, H8)
    xp = jnp.concatenate(
        [x.reshape(N * D // 128, 128),
         jnp.zeros((NPK - N * D // 128, 128), jnp.float32)])

    xs_p = _sc_gather(x, src3).reshape(EPK, 128)
    msg_p = _tc_edge(ea_p, xs_p, w1rb, smatb, w2fb)
    part_p = _sc_scatter(msg_p.reshape(NW, EPT, D), dst3,
                         zeros_slab).reshape(NC, NPK, 128)

    out_p = _tc_node(xp, part_p, eps.astype(jnp.float32),
                     nw1blk, b1t, nw2blk, b2t, mublk, g8, b8)
    return out_p.reshape(NPAD, H)[:N]


# speedup vs baseline: 1.6000x; 1.0465x over previous
import jax
import jax.numpy as jnp
from jax import lax
from jax.experimental import pallas as pl
from jax.experimental.pallas import tpu as pltpu
from jax.experimental.pallas import tpu_sc as plsc

N = 10000
E = 160000
D = 16
OUT = 16
H = 32

NC = 2
NS = 16
NW = NC * NS
CH = 125
NCHUNK = 40
EPT = CH * NCHUNK
EP8 = EPT // 8
EPK = E // 8
NROW = 640
NPAD = NROW * NS
NPK = NPAD // 8
DEPTH = 16


def _mesh():
    return plsc.VectorSubcoreMesh(
        core_axis_name="c", subcore_axis_name="s", num_cores=NC, num_subcores=NS)


def _gather_body(x_hbm, src_hbm, xs_hbm, idx_v, rows_v, sem, xsm):
    c = lax.axis_index("c")
    s = lax.axis_index("s")
    w = s * NC + c

    @pl.when(s == 0)
    def _():
        pltpu.sync_copy(x_hbm, xsm)
    pltpu.sync_copy(src_hbm.at[w], idx_v)
    plsc.subcore_barrier()

    def fire(j):
        pltpu.async_copy(xsm.at[idx_v.at[j]],
                         rows_v.at[pl.ds(j * CH, CH)], sem)

    def drain(j):
        pltpu.make_async_copy(xsm.at[idx_v.at[j]],
                              rows_v.at[pl.ds(j * CH, CH)], sem).wait()

    lax.fori_loop(0, DEPTH, lambda j, v: (fire(j), v)[1], 0)

    def steady(j, v):
        @pl.when(j + DEPTH < NCHUNK)
        def _():
            fire(j + DEPTH)
        drain(j)
        return v

    lax.fori_loop(0, NCHUNK, steady, 0)
    pltpu.sync_copy(rows_v, xs_hbm.at[w])


@jax.jit
def _sc_gather(x, src3):
    k = pl.kernel(
        _gather_body,
        out_type=jax.ShapeDtypeStruct((NW, EPT, D), jnp.float32),
        mesh=_mesh(),
        compiler_params=pltpu.CompilerParams(use_tc_tiling_on_sc=False),
        scratch_types=[
            pltpu.VMEM((NCHUNK, CH), jnp.int32),
            pltpu.VMEM((EPT, D), jnp.float32),
            pltpu.SemaphoreType.DMA,
            pltpu.VMEM_SHARED((N, D), jnp.float32),
        ],
    )
    return k(x, src3)


def _scatter_body(msg_hbm, dst_hbm, zeros_hbm, part_hbm, idx_v, msg_v, sem, acc):
    c = lax.axis_index("c")
    s = lax.axis_index("s")
    w = s * NC + c
    pltpu.sync_copy(zeros_hbm, acc.at[pl.ds(s * NROW, NROW)])
    pltpu.sync_copy(dst_hbm.at[w], idx_v)
    pltpu.sync_copy(msg_hbm.at[w], msg_v)
    plsc.subcore_barrier()

    def fire(j):
        pltpu.async_copy(msg_v.at[pl.ds(j * CH, CH)],
                         acc.at[idx_v.at[j]], sem, add=True)

    def drain(j):
        pltpu.make_async_copy(msg_v.at[pl.ds(j * CH, CH)],
                              acc.at[idx_v.at[j]], sem).wait()

    lax.fori_loop(0, DEPTH, lambda j, v: (fire(j), v)[1], 0)

    def steady(j, v):
        @pl.when(j + DEPTH < NCHUNK)
        def _():
            fire(j + DEPTH)
        drain(j)
        return v

    lax.fori_loop(0, NCHUNK, steady, 0)
    plsc.subcore_barrier()
    pltpu.sync_copy(acc.at[pl.ds(s * NROW, NROW)],
                    part_hbm.at[c, pl.ds(s * NROW, NROW)])


@jax.jit
def _sc_scatter(msg, dst3, zeros_slab):
    k = pl.kernel(
        _scatter_body,
        out_type=jax.ShapeDtypeStruct((NC, NPAD, D), jnp.float32),
        mesh=_mesh(),
        compiler_params=pltpu.CompilerParams(use_tc_tiling_on_sc=False),
        scratch_types=[
            pltpu.VMEM((NCHUNK, CH), jnp.int32),
            pltpu.VMEM((EPT, D), jnp.float32),
            pltpu.SemaphoreType.DMA,
            pltpu.VMEM_SHARED((NPAD, D), jnp.float32),
        ],
    )
    return k(msg, dst3, zeros_slab)


BR = 2000


def _edge_body(ea_ref, xs_ref, w1r_ref, s_ref, w2f_ref, msg_ref):
    ea = ea_ref[...].astype(jnp.bfloat16)
    xs = xs_ref[...]
    for e8 in range(8):
        sl = slice(e8 * D, (e8 + 1) * D)
        t8 = jnp.maximum(jnp.dot(ea[:, sl], w1r_ref[...],
                                 preferred_element_type=jnp.float32), 0.0)
        x8 = jnp.tile(xs[:, sl], (1, D))
        u8 = (t8 * x8).astype(jnp.bfloat16)
        msg_ref[:, sl] = jnp.dot(u8, w2f_ref[...],
                                 preferred_element_type=jnp.float32)


def _tc_edge(ea_p, xs_p, w1r, smat, w2f):
    nblk = EPK // BR
    return pl.pallas_call(
        _edge_body,
        grid=(nblk,),
        in_specs=[
            pl.BlockSpec((BR, 128), lambda i: (i, 0)),
            pl.BlockSpec((BR, 128), lambda i: (i, 0)),
            pl.BlockSpec((D, D * OUT), lambda i: (0, 0)),
            pl.BlockSpec((D, D * OUT), lambda i: (0, 0)),
            pl.BlockSpec((D * OUT, OUT), lambda i: (0, 0)),
        ],
        out_specs=pl.BlockSpec((BR, 128), lambda i: (i, 0)),
        out_shape=jax.ShapeDtypeStruct((EPK, 128), jnp.float32),
    )(ea_p, xs_p, w1r, smat, w2f)


H8 = 8 * H


def _node_body(xp_ref, part_ref, eps_ref, w1_ref, b1_ref, w2_ref, b2_ref,
               mu_ref, g_ref, b_ref, out_ref):
    hf = (1.0 + eps_ref[0]) * (xp_ref[...] + part_ref[0] + part_ref[1])
    z = jnp.dot(hf, w1_ref[...], preferred_element_type=jnp.float32) + b1_ref[...]
    z = jnp.where(z > 0, z, 0.01 * z)
    z = jnp.dot(z, w2_ref[...], preferred_element_type=jnp.float32) + b2_ref[...]
    mu = jnp.dot(z, mu_ref[...], preferred_element_type=jnp.float32)
    zc = z - mu
    var = jnp.dot(zc * zc, mu_ref[...], preferred_element_type=jnp.float32)
    out_ref[...] = zc * jax.lax.rsqrt(var + 1e-5) * g_ref[...] + b_ref[...]


def _tc_node(xp, part_p, eps, w1blk, b1t, w2blk, b2t, mublk, g8, b8):
    return pl.pallas_call(
        _node_body,
        grid=(1,),
        in_specs=[
            pl.BlockSpec((NPK, 128), lambda i: (0, 0)),
            pl.BlockSpec((NC, NPK, 128), lambda i: (0, 0, 0)),
            pl.BlockSpec((1,), lambda i: (0,)),
            pl.BlockSpec((128, H8), lambda i: (0, 0)),
            pl.BlockSpec((1, H8), lambda i: (0, 0)),
            pl.BlockSpec((H8, H8), lambda i: (0, 0)),
            pl.BlockSpec((1, H8), lambda i: (0, 0)),
            pl.BlockSpec((H8, H8), lambda i: (0, 0)),
            pl.BlockSpec((1, H8), lambda i: (0, 0)),
            pl.BlockSpec((1, H8), lambda i: (0, 0)),
        ],
        out_specs=pl.BlockSpec((NPK, H8), lambda i: (0, 0)),
        out_shape=jax.ShapeDtypeStruct((NPK, H8), jnp.float32),
    )(xp, part_p, eps, w1blk, b1t, w2blk, b2t, mublk, g8, b8)


def kernel(x, edge_index, edge_attr, eps, W_e1, W_e2, mlp_W1, mlp_b1,
           mlp_W2, mlp_b2, ln_g, ln_b):
    ea_p = edge_attr.reshape(EPK, 128)
    src3 = edge_index[0].reshape(NW, NCHUNK, CH)
    dst3 = edge_index[1].reshape(NW, NCHUNK, CH)

    eye = jnp.eye(D, dtype=jnp.float32)
    eye8 = jnp.eye(8, dtype=jnp.float32)
    rmat = jnp.repeat(eye, D, axis=1)
    smat = jnp.tile(eye, (1, D))
    w1rb = (W_e1 @ rmat).astype(jnp.bfloat16)
    smatb = smat.astype(jnp.bfloat16)
    w2fb = W_e2.reshape(D * D, OUT).astype(jnp.bfloat16)
    zeros_slab = jnp.zeros((NROW, D), jnp.float32)
    nw1blk = jnp.kron(eye8, mlp_W1)
    nw2blk = jnp.kron(eye8, mlp_W2)
    mublk = jnp.kron(eye8, jnp.full((H, H), 1.0 / H, jnp.float32))
    b1t = jnp.tile(mlp_b1, 8).reshape(1, H8)
    b2t = jnp.tile(mlp_b2, 8).reshape(1, H8)
    g8 = jnp.tile(ln_g, 8).reshape(1, H8)
    b8 = jnp.tile(ln_b, 8).reshape(1, H8)
    xp = jnp.concatenate(
        [x.reshape(N * D // 128, 128),
         jnp.zeros((NPK - N * D // 128, 128), jnp.float32)])

    xs_p = _sc_gather(x, src3).reshape(EPK, 128)
    msg_p = _tc_edge(ea_p, xs_p, w1rb, smatb, w2fb)
    part_p = _sc_scatter(msg_p.reshape(NW, EPT, D), dst3,
                         zeros_slab).reshape(NC, NPK, 128)

    out_p = _tc_node(xp, part_p, eps.astype(jnp.float32),
                     nw1blk, b1t, nw2blk, b2t, mublk, g8, b8)
    return out_p.reshape(NPAD, H)[:N]


# speedup vs baseline: 1.6038x; 1.0023x over previous
import jax
import jax.numpy as jnp
from jax import lax
from jax.experimental import pallas as pl
from jax.experimental.pallas import tpu as pltpu
from jax.experimental.pallas import tpu_sc as plsc

N = 10000
E = 160000
D = 16
OUT = 16
H = 32

NC = 2
NS = 16
NW = NC * NS
CH = 125
NCHUNK = 40
EPT = CH * NCHUNK
EP8 = EPT // 8
EPK = E // 8
NROW = 640
NPAD = NROW * NS
NPK = NPAD // 8
DEPTH = 16


def _mesh():
    return plsc.VectorSubcoreMesh(
        core_axis_name="c", subcore_axis_name="s", num_cores=NC, num_subcores=NS)


def _gather_body(x_hbm, src_hbm, xs_hbm, idx_v, rows_v, sem, xsm):
    c = lax.axis_index("c")
    s = lax.axis_index("s")
    w = s * NC + c

    @pl.when(s == 0)
    def _():
        pltpu.sync_copy(x_hbm, xsm)
    pltpu.sync_copy(src_hbm.at[w], idx_v)
    plsc.subcore_barrier()

    def fire(j):
        pltpu.async_copy(xsm.at[idx_v.at[j]],
                         rows_v.at[pl.ds(j * CH, CH)], sem)

    def drain(j):
        pltpu.make_async_copy(xsm.at[idx_v.at[j]],
                              rows_v.at[pl.ds(j * CH, CH)], sem).wait()

    lax.fori_loop(0, DEPTH, lambda j, v: (fire(j), v)[1], 0)

    def steady(j, v):
        @pl.when(j + DEPTH < NCHUNK)
        def _():
            fire(j + DEPTH)
        drain(j)
        return v

    lax.fori_loop(0, NCHUNK, steady, 0)
    pltpu.sync_copy(rows_v, xs_hbm.at[w])


@jax.jit
def _sc_gather(x, src3):
    k = pl.kernel(
        _gather_body,
        out_type=jax.ShapeDtypeStruct((NW, EPT, D), jnp.float32),
        mesh=_mesh(),
        compiler_params=pltpu.CompilerParams(use_tc_tiling_on_sc=False),
        scratch_types=[
            pltpu.VMEM((NCHUNK, CH), jnp.int32),
            pltpu.VMEM((EPT, D), jnp.float32),
            pltpu.SemaphoreType.DMA,
            pltpu.VMEM_SHARED((N, D), jnp.float32),
        ],
    )
    return k(x, src3)


def _scatter_body(msg_hbm, dst_hbm, zeros_hbm, part_hbm, idx_v, msg_v, sem, acc):
    c = lax.axis_index("c")
    s = lax.axis_index("s")
    w = s * NC + c
    pltpu.sync_copy(zeros_hbm, acc.at[pl.ds(s * NROW, NROW)])
    pltpu.sync_copy(dst_hbm.at[w], idx_v)
    pltpu.sync_copy(msg_hbm.at[w], msg_v)
    plsc.subcore_barrier()

    def fire(j):
        pltpu.async_copy(msg_v.at[pl.ds(j * CH, CH)],
                         acc.at[idx_v.at[j]], sem, add=True)

    def drain(j):
        pltpu.make_async_copy(msg_v.at[pl.ds(j * CH, CH)],
                              acc.at[idx_v.at[j]], sem).wait()

    lax.fori_loop(0, DEPTH, lambda j, v: (fire(j), v)[1], 0)

    def steady(j, v):
        @pl.when(j + DEPTH < NCHUNK)
        def _():
            fire(j + DEPTH)
        drain(j)
        return v

    lax.fori_loop(0, NCHUNK, steady, 0)
    plsc.subcore_barrier()
    pltpu.sync_copy(acc.at[pl.ds(s * NROW, NROW)],
                    part_hbm.at[c, pl.ds(s * NROW, NROW)])


@jax.jit
def _sc_scatter(msg, dst3, zeros_slab):
    k = pl.kernel(
        _scatter_body,
        out_type=jax.ShapeDtypeStruct((NC, NPAD, D), jnp.float32),
        mesh=_mesh(),
        compiler_params=pltpu.CompilerParams(use_tc_tiling_on_sc=False),
        scratch_types=[
            pltpu.VMEM((NCHUNK, CH), jnp.int32),
            pltpu.VMEM((EPT, D), jnp.float32),
            pltpu.SemaphoreType.DMA,
            pltpu.VMEM_SHARED((NPAD, D), jnp.float32),
        ],
    )
    return k(msg, dst3, zeros_slab)


BR = 4000


def _edge_body(ea_ref, xs_ref, w1r_ref, s_ref, w2f_ref, msg_ref):
    ea = ea_ref[...].astype(jnp.bfloat16)
    xs = xs_ref[...]
    for e8 in range(8):
        sl = slice(e8 * D, (e8 + 1) * D)
        t8 = jnp.maximum(jnp.dot(ea[:, sl], w1r_ref[...],
                                 preferred_element_type=jnp.float32), 0.0)
        x8 = jnp.tile(xs[:, sl], (1, D))
        u8 = (t8 * x8).astype(jnp.bfloat16)
        msg_ref[:, sl] = jnp.dot(u8, w2f_ref[...],
                                 preferred_element_type=jnp.float32)


def _tc_edge(ea_p, xs_p, w1r, smat, w2f):
    nblk = EPK // BR
    return pl.pallas_call(
        _edge_body,
        grid=(nblk,),
        in_specs=[
            pl.BlockSpec((BR, 128), lambda i: (i, 0)),
            pl.BlockSpec((BR, 128), lambda i: (i, 0)),
            pl.BlockSpec((D, D * OUT), lambda i: (0, 0)),
            pl.BlockSpec((D, D * OUT), lambda i: (0, 0)),
            pl.BlockSpec((D * OUT, OUT), lambda i: (0, 0)),
        ],
        out_specs=pl.BlockSpec((BR, 128), lambda i: (i, 0)),
        out_shape=jax.ShapeDtypeStruct((EPK, 128), jnp.float32),
    )(ea_p, xs_p, w1r, smat, w2f)


H8 = 8 * H


def _node_body(xp_ref, part_ref, eps_ref, w1_ref, b1_ref, w2_ref, b2_ref,
               mu_ref, g_ref, b_ref, out_ref):
    hf = (1.0 + eps_ref[0]) * (xp_ref[...] + part_ref[0] + part_ref[1])
    z = jnp.dot(hf, w1_ref[...], preferred_element_type=jnp.float32) + b1_ref[...]
    z = jnp.where(z > 0, z, 0.01 * z)
    z = jnp.dot(z, w2_ref[...], preferred_element_type=jnp.float32) + b2_ref[...]
    mu = jnp.dot(z, mu_ref[...], preferred_element_type=jnp.float32)
    zc = z - mu
    var = jnp.dot(zc * zc, mu_ref[...], preferred_element_type=jnp.float32)
    out_ref[...] = zc * jax.lax.rsqrt(var + 1e-5) * g_ref[...] + b_ref[...]


def _tc_node(xp, part_p, eps, w1blk, b1t, w2blk, b2t, mublk, g8, b8):
    return pl.pallas_call(
        _node_body,
        grid=(1,),
        in_specs=[
            pl.BlockSpec((NPK, 128), lambda i: (0, 0)),
            pl.BlockSpec((NC, NPK, 128), lambda i: (0, 0, 0)),
            pl.BlockSpec((1,), lambda i: (0,)),
            pl.BlockSpec((128, H8), lambda i: (0, 0)),
            pl.BlockSpec((1, H8), lambda i: (0, 0)),
            pl.BlockSpec((H8, H8), lambda i: (0, 0)),
            pl.BlockSpec((1, H8), lambda i: (0, 0)),
            pl.BlockSpec((H8, H8), lambda i: (0, 0)),
            pl.BlockSpec((1, H8), lambda i: (0, 0)),
            pl.BlockSpec((1, H8), lambda i: (0, 0)),
        ],
        out_specs=pl.BlockSpec((NPK, H8), lambda i: (0, 0)),
        out_shape=jax.ShapeDtypeStruct((NPK, H8), jnp.float32),
    )(xp, part_p, eps, w1blk, b1t, w2blk, b2t, mublk, g8, b8)


def kernel(x, edge_index, edge_attr, eps, W_e1, W_e2, mlp_W1, mlp_b1,
           mlp_W2, mlp_b2, ln_g, ln_b):
    ea_p = edge_attr.reshape(EPK, 128)
    src3 = edge_index[0].reshape(NW, NCHUNK, CH)
    dst3 = edge_index[1].reshape(NW, NCHUNK, CH)

    eye = jnp.eye(D, dtype=jnp.float32)
    eye8 = jnp.eye(8, dtype=jnp.float32)
    rmat = jnp.repeat(eye, D, axis=1)
    smat = jnp.tile(eye, (1, D))
    w1rb = (W_e1 @ rmat).astype(jnp.bfloat16)
    smatb = smat.astype(jnp.bfloat16)
    w2fb = W_e2.reshape(D * D, OUT).astype(jnp.bfloat16)
    zeros_slab = jnp.zeros((NROW, D), jnp.float32)
    nw1blk = jnp.kron(eye8, mlp_W1)
    nw2blk = jnp.kron(eye8, mlp_W2)
    mublk = jnp.kron(eye8, jnp.full((H, H), 1.0 / H, jnp.float32))
    b1t = jnp.tile(mlp_b1, 8).reshape(1, H8)
    b2t = jnp.tile(mlp_b2, 8).reshape(1, H8)
    g8 = jnp.tile(ln_g, 8).reshape(1, H8)
    b8 = jnp.tile(ln_b, 8).reshape(1, H8)
    xp = jnp.concatenate(
        [x.reshape(N * D // 128, 128),
         jnp.zeros((NPK - N * D // 128, 128), jnp.float32)])

    xs_p = _sc_gather(x, src3).reshape(EPK, 128)
    msg_p = _tc_edge(ea_p, xs_p, w1rb, smatb, w2fb)
    part_p = _sc_scatter(msg_p.reshape(NW, EPT, D), dst3,
                         zeros_slab).reshape(NC, NPK, 128)

    out_p = _tc_node(xp, part_p, eps.astype(jnp.float32),
                     nw1blk, b1t, nw2blk, b2t, mublk, g8, b8)
    return out_p.reshape(NPAD, H)[:N]
